# baseline (speedup 1.0000x reference)
import functools

import jax
import jax.numpy as jnp
from jax import lax
from jax.experimental import pallas as pl
from jax.experimental.pallas import tpu as pltpu
from jax.experimental.pallas import tpu_sc as plsc

VOCAB = 8192
EMB_D = 8192
NUM_CORES = 2
NUM_SUBCORES = 16
NW = NUM_CORES * NUM_SUBCORES
TOKENS = 4 * 2048
BPW = TOKENS // NW
SBT = 2
NSB = BPW // SBT
NREG = 4
ITER_SB = 8
NITER = NSB // ITER_SB

_mesh = plsc.VectorSubcoreMesh(core_axis_name="c", subcore_axis_name="s")


@functools.partial(
    pl.kernel,
    mesh=_mesh,
    out_type=jax.ShapeDtypeStruct((TOKENS, EMB_D), jnp.float32),
    scratch_types=(
        [pltpu.VMEM((BPW, 1), jnp.int32),
         pltpu.VMEM((NSB + 8,), jnp.int32),
         pltpu.VMEM((BPW,), jnp.int32),
         pltpu.VMEM((BPW,), jnp.int32),
         pltpu.VMEM((NREG * SBT, 1, EMB_D), jnp.float32)]
        + [pltpu.SemaphoreType.DMA for _ in range(2 * NREG)]
    ),
)
def _emb_gather(uq_hbm, cnt_hbm, src_hbm, dst_hbm, table_hbm, out_hbm,
                uq_v, cnt_v, src_v, dst_v, bufs, *sems):
    gsems = sems[:NREG]
    wsems = sems[NREG:]
    wid = lax.axis_index("s") * NUM_CORES + lax.axis_index("c")
    pltpu.sync_copy(uq_hbm.at[wid], uq_v)
    pltpu.sync_copy(cnt_hbm.at[wid], cnt_v)
    pltpu.sync_copy(src_hbm.at[wid], src_v)
    pltpu.sync_copy(dst_hbm.at[wid], dst_v)

    def gdesc(s, j, r):
        return pltpu.make_async_copy(
            table_hbm.at[uq_v.at[SBT * s + j]], bufs.at[r * SBT + j],
            gsems[r])

    def write_start(row, dstpos, u):
        pltpu.make_async_copy(
            bufs.at[row], out_hbm.at[pl.ds(dstpos, 1)], wsems[u]).start()

    def write_drain(x):
        pltpu.make_async_copy(
            bufs.at[0], out_hbm.at[pl.ds(0, 1)], wsems[x]).wait()

    def sb_block(s, b, cnt16, src16, dst16, has_prev, live_next):
        u = b % NREG
        cs = cnt16[b]
        for j in range(SBT):
            pl.when(j < cs)(lambda j=j: gdesc(s, j, u).wait())
        for v in range(SBT):
            lane = SBT * b + v
            write_start(src16[lane], dst16[lane], u)
        if has_prev:
            for _ in range(SBT):
                write_drain((u + 3) % NREG)
        if live_next:
            cn = cnt16[b + 2]
            for j in range(SBT):
                pl.when(j < cn)(
                    lambda j=j: gdesc(s + 2, j, (u + 2) % NREG).start())

    def load_meta(s0):
        return (cnt_v[pl.ds(s0, 16)],
                src_v[pl.ds(SBT * s0, 16)],
                dst_v[pl.ds(SBT * s0, 16)])

    cnt16, src16, dst16 = load_meta(0)
    for j in range(SBT):
        pl.when(j < cnt16[0])(lambda j=j: gdesc(0, j, 0).start())
        pl.when(j < cnt16[1])(lambda j=j: gdesc(1, j, 1).start())
    for b in range(ITER_SB):
        sb_block(b, b, cnt16, src16, dst16, b >= 1, True)

    def group(g, carry):
        s0 = g * ITER_SB
        c16, s16, d16 = load_meta(s0)
        for b in range(ITER_SB):
            sb_block(s0 + b, b, c16, s16, d16, True, True)
        return carry

    lax.fori_loop(1, NITER - 1, group, 0)

    s0t = (NITER - 1) * ITER_SB
    cnt16, src16, dst16 = load_meta(s0t)
    for b in range(ITER_SB):
        sb_block(s0t + b, b, cnt16, src16, dst16, True, s0t + b + 2 < NSB)
    for _ in range(SBT):
        write_drain((NSB - 1) % NREG)


def _metadata(flat):
    order = jnp.argsort(flat)
    sidx = jnp.take(flat, order)
    pos = jnp.arange(TOKENS, dtype=jnp.int32)
    sb = pos // SBT
    chg = jnp.concatenate(
        [jnp.ones((1,), bool), sidx[1:] != sidx[:-1]])
    new0 = chg | ((pos % BPW) == 0)
    rs = lax.cummax(jnp.where(new0, pos, -1), axis=0)
    d = sb - rs // SBT
    new = jnp.ones_like(new0)
    gp = lax.cummax(jnp.where(new, pos, -1), axis=0)
    new_i = new.astype(jnp.int32)
    new_v0 = new_i[0::SBT]
    rank = jnp.stack(
        [jnp.zeros_like(new_v0), new_v0], axis=1).reshape(-1)
    cnt = new_i.reshape(-1, SBT).sum(axis=1).astype(jnp.int32)
    cnt = jnp.pad(cnt.reshape(NW, NSB), ((0, 0), (0, 8)))
    sgp = gp // SBT
    srcrow = ((sgp % NREG) * SBT + jnp.take(rank, gp)).astype(jnp.int32)
    slot = jnp.where(new, SBT * sb + rank, TOKENS)
    uq = jnp.zeros((TOKENS,), jnp.int32).at[slot].set(sidx, mode="drop")
    return (uq.reshape(NW, BPW, 1), cnt,
            srcrow.reshape(NW, BPW), order.astype(jnp.int32).reshape(NW, BPW))


def kernel(input_ids, embedding_weight):
    batch, seq = input_ids.shape
    flat = input_ids.reshape(-1).astype(jnp.int32) % VOCAB
    uq, cnt, src, dst = _metadata(flat)
    out = _emb_gather(uq, cnt, src, dst, embedding_weight)
    return out.reshape(batch, seq, EMB_D)


# speedup vs baseline: 1.2949x; 1.2949x over previous
import functools

import jax
import jax.numpy as jnp
from jax import lax
from jax.experimental import pallas as pl
from jax.experimental.pallas import tpu as pltpu
from jax.experimental.pallas import tpu_sc as plsc

VOCAB = 8192
EMB_D = 8192
NUM_CORES = 2
NUM_SUBCORES = 16
NW = NUM_CORES * NUM_SUBCORES
TOKENS = 4 * 2048
BPW = TOKENS // NW
K = 2
NBUF = 4
NCHUNK = BPW // K
NGROUP = NCHUNK // NBUF

_mesh = plsc.VectorSubcoreMesh(core_axis_name="c", subcore_axis_name="s")


@functools.partial(
    pl.kernel,
    mesh=_mesh,
    out_type=jax.ShapeDtypeStruct((TOKENS, EMB_D), jnp.float32),
    scratch_types=(
        [pltpu.VMEM((NCHUNK, K), jnp.int32)]
        + [pltpu.VMEM((K, EMB_D), jnp.float32) for _ in range(NBUF)]
        + [pltpu.SemaphoreType.DMA for _ in range(2 * NBUF)]
    ),
)
def _emb_gather(idx_hbm, table_hbm, out_hbm, idx_v, *rest):
    bufs = rest[:NBUF]
    gsems = rest[NBUF:2 * NBUF]
    wsems = rest[2 * NBUF:]
    wid = lax.axis_index("s") * NUM_CORES + lax.axis_index("c")
    base = wid * BPW
    pltpu.sync_copy(idx_hbm.at[wid], idx_v)

    def gather_copy(j, u):
        return pltpu.make_async_copy(
            table_hbm.at[idx_v.at[j]], bufs[u], gsems[u])

    def write_copy(j, u):
        return pltpu.make_async_copy(
            bufs[u], out_hbm.at[pl.ds(base + j * K, K)], wsems[u])

    def step(j, u, first, live_next):
        gather_copy(j, u).wait()
        write_copy(j, u).start()
        if not first:
            write_copy(j - 1, (u - 1) % NBUF).wait()
        if live_next:
            gather_copy(j + NBUF - 1, (u - 1) % NBUF).start()

    for u in range(NBUF - 1):
        gather_copy(u, u).start()
    for u in range(NBUF):
        step(u, u, u == 0, True)

    def group(g, carry):
        for u in range(NBUF):
            step(g * NBUF + u, u, False, True)
        return carry

    lax.fori_loop(1, NGROUP - 1, group, 0)

    for u in range(NBUF):
        j = (NGROUP - 1) * NBUF + u
        step(j, u, False, u == 0)
    write_copy(NCHUNK - 1, (NCHUNK - 1) % NBUF).wait()


def kernel(input_ids, embedding_weight):
    batch, seq = input_ids.shape
    idx = (input_ids.astype(jnp.int32) % VOCAB).reshape(NW, NCHUNK, K)
    out = _emb_gather(idx, embedding_weight)
    return out.reshape(batch, seq, EMB_D)
